# baseline (speedup 1.0000x reference)
import jax
import jax.numpy as jnp
from jax.experimental import pallas as pl

_NUM_CLASSES = 1000
_LS = 0.1
_COLD = _LS / (_NUM_CLASSES - 1)
_HOT = (1.0 - _LS) + _COLD

_B_BLK = 8


def _onehot_body(x_ref, o_ref):
    idx = x_ref[...]
    iota = jax.lax.broadcasted_iota(
        jnp.int32, (idx.shape[0], idx.shape[1], _NUM_CLASSES), 2
    )
    o_ref[...] = jnp.where(
        idx[:, :, None] == iota,
        jnp.float32(_HOT),
        jnp.float32(_COLD),
    )


def kernel(x_i):
    b, s = x_i.shape
    return pl.pallas_call(
        _onehot_body,
        grid=(b // _B_BLK,),
        in_specs=[pl.BlockSpec((_B_BLK, s), lambda i: (i, 0))],
        out_specs=pl.BlockSpec((_B_BLK, s, _NUM_CLASSES), lambda i: (i, 0, 0)),
        out_shape=jax.ShapeDtypeStruct((b, s, _NUM_CLASSES), jnp.float32),
    )(x_i)
